# baseline (speedup 1.0000x reference)
import jax
import jax.numpy as jnp
from jax import lax
from jax.experimental import pallas as pl
from jax.experimental.pallas import tpu as pltpu
from jax.experimental.pallas import tpu_sc as plsc

N_HEADS = 16
NQ = 1024
NK = 1024
MAX_SIZE = 256
W = 63
WIN = W * W
HSTRIDE = 4096
LO = (MAX_SIZE - 1) - 31
BASE0 = 31 * W + 31

_info = plsc.get_sparse_core_info()
NC, NS, L = _info.num_cores, _info.num_subcores, _info.num_lanes
NW = NC * NS
QPW = NQ // NW


def _body(qx_hbm, qy_hbm, kx_hbm, ky_hbm, tab_hbm, out_hbm, *refs):
    tabs = refs[:N_HEADS]
    kx_v, ky_v, off_v, qx_v, qy_v, base_v, buf0, buf1, sem0, sem1 = refs[N_HEADS:]
    wid = lax.axis_index("s") * NC + lax.axis_index("c")
    q0 = wid * QPW

    for h in range(N_HEADS):
        pltpu.sync_copy(tab_hbm.at[h], tabs[h])
    pltpu.sync_copy(kx_hbm, kx_v)
    pltpu.sync_copy(ky_hbm, ky_v)
    pltpu.sync_copy(qx_hbm.at[pl.ds(q0, QPW)], qx_v)
    pltpu.sync_copy(qy_hbm.at[pl.ds(q0, QPW)], qy_v)

    def _off(j, c):
        s = pl.ds(pl.multiple_of(j * L, L), L)
        off_v[s] = kx_v[s] * W + ky_v[s]
        return c
    lax.fori_loop(0, NK // L, _off, 0)

    def _base(j, c):
        s = pl.ds(pl.multiple_of(j * L, L), L)
        base_v[s] = qx_v[s] * W + qy_v[s] + BASE0
        return c
    lax.fori_loop(0, QPW // L, _base, 0)

    def _compute(q, buf):
        bs = plsc.load_gather(base_v, [jnp.full((L,), q, jnp.int32)])

        for j in range(NK // L):
            s = pl.ds(j * L, L)
            idx0 = bs - off_v[s]
            for h in range(N_HEADS):
                buf[h, 0, s] = plsc.load_gather(tabs[h], [idx0])

    def _start(q, buf, sem):
        pltpu.async_copy(buf, out_hbm.at[:, pl.ds(q0 + q, 1), :], sem)

    def _wait(q, buf, sem):
        pltpu.make_async_copy(buf, out_hbm.at[:, pl.ds(q0 + q, 1), :], sem).wait()

    def _t(t, c):
        qa = 2 * t
        qb = qa + 1

        @pl.when(t > 0)
        def _():
            _wait(qa - 2, buf0, sem0)
        _compute(qa, buf0)
        _start(qa, buf0, sem0)

        @pl.when(t > 0)
        def _():
            _wait(qb - 2, buf1, sem1)
        _compute(qb, buf1)
        _start(qb, buf1, sem1)
        return c
    lax.fori_loop(0, QPW // 2, _t, 0)
    _wait(QPW - 2, buf0, sem0)
    _wait(QPW - 1, buf1, sem1)


def kernel(coords_q, coords_k, bias_table):
    qx = coords_q[:, 0]
    qy = coords_q[:, 1]
    kx = coords_k[:, 0]
    ky = coords_k[:, 1]
    win = bias_table[:, LO:LO + W, LO:LO + W].reshape(N_HEADS, WIN)
    tab = jnp.pad(win, ((0, 0), (0, HSTRIDE - WIN)))

    mesh = plsc.VectorSubcoreMesh(core_axis_name="c", subcore_axis_name="s")
    f = pl.kernel(
        _body,
        out_type=jax.ShapeDtypeStruct((N_HEADS, NQ, NK), jnp.float32),
        mesh=mesh,
        compiler_params=pltpu.CompilerParams(needs_layout_passes=False),
        scratch_types=(
            [pltpu.VMEM((HSTRIDE,), jnp.float32) for _ in range(N_HEADS)]
            + [
                pltpu.VMEM((NK,), jnp.int32),
                pltpu.VMEM((NK,), jnp.int32),
                pltpu.VMEM((NK,), jnp.int32),
                pltpu.VMEM((QPW,), jnp.int32),
                pltpu.VMEM((QPW,), jnp.int32),
                pltpu.VMEM((QPW,), jnp.int32),
                pltpu.VMEM((N_HEADS, 1, NK), jnp.float32),
                pltpu.VMEM((N_HEADS, 1, NK), jnp.float32),
                pltpu.SemaphoreType.DMA,
                pltpu.SemaphoreType.DMA,
            ]
        ),
    )
    return f(qx, qy, kx, ky, tab)


# speedup vs baseline: 3.4549x; 3.4549x over previous
import jax
import jax.numpy as jnp
from jax import lax
from jax.experimental import pallas as pl
from jax.experimental.pallas import tpu as pltpu
from jax.experimental.pallas import tpu_sc as plsc

N_HEADS = 16
NQ = 1024
NK = 1024
MAX_SIZE = 256
W = 63
WIN = W * W
HSTRIDE = 4096
LO = (MAX_SIZE - 1) - 31
BASE0 = 31 * W + 31

_info = plsc.get_sparse_core_info()
NC, NS, L = _info.num_cores, _info.num_subcores, _info.num_lanes
NW = NC * NS
QPW = NQ // NW


def _body(qx_hbm, qy_hbm, kx_hbm, ky_hbm, tab_hbm, out_hbm,
          tab_v, kx_v, ky_v, off_v, qx_v, qy_v, base_v, buf0, buf1,
          sem_in, sem0, sem1):
    wid = lax.axis_index("s") * NC + lax.axis_index("c")
    q0 = wid * QPW

    cps = [
        pltpu.async_copy(tab_hbm, tab_v, sem_in),
        pltpu.async_copy(kx_hbm, kx_v, sem_in),
        pltpu.async_copy(ky_hbm, ky_v, sem_in),
        pltpu.async_copy(qx_hbm.at[pl.ds(q0, QPW)], qx_v, sem_in),
        pltpu.async_copy(qy_hbm.at[pl.ds(q0, QPW)], qy_v, sem_in),
    ]
    for cp in cps:
        cp.wait()

    def _off(j, c):
        s = pl.ds(pl.multiple_of(j * L, L), L)
        off_v[s] = kx_v[s] * W + ky_v[s]
        return c
    lax.fori_loop(0, NK // L, _off, 0)

    def _base(j, c):
        s = pl.ds(pl.multiple_of(j * L, L), L)
        base_v[s] = qx_v[s] * W + qy_v[s] + BASE0
        return c
    lax.fori_loop(0, QPW // L, _base, 0)

    def _compute(q, buf):
        bs = plsc.load_gather(base_v, [jnp.full((L,), q, jnp.int32)])

        @plsc.parallel_loop(0, NK, L, unroll=4)
        def _j(i):
            s = pl.ds(pl.multiple_of(i, L), L)
            idx0 = bs - off_v[s]
            for h in range(N_HEADS):
                buf[h, 0, s] = plsc.load_gather(tab_v, [idx0 + h * HSTRIDE])

    def _start(q, buf, sem):
        pltpu.async_copy(buf, out_hbm.at[:, pl.ds(q0 + q, 1), :], sem)

    def _wait(q, buf, sem):
        pltpu.make_async_copy(buf, out_hbm.at[:, pl.ds(q0 + q, 1), :], sem).wait()

    def _t(t, c):
        qa = 2 * t
        qb = qa + 1

        @pl.when(t > 0)
        def _():
            _wait(qa - 2, buf0, sem0)
        _compute(qa, buf0)
        _start(qa, buf0, sem0)

        @pl.when(t > 0)
        def _():
            _wait(qb - 2, buf1, sem1)
        _compute(qb, buf1)
        _start(qb, buf1, sem1)
        return c
    lax.fori_loop(0, QPW // 2, _t, 0)
    _wait(QPW - 2, buf0, sem0)
    _wait(QPW - 1, buf1, sem1)


def kernel(coords_q, coords_k, bias_table):
    qx = coords_q[:, 0]
    qy = coords_q[:, 1]
    kx = coords_k[:, 0]
    ky = coords_k[:, 1]
    win = bias_table[:, LO:LO + W, LO:LO + W].reshape(N_HEADS, WIN)
    tab = jnp.pad(win, ((0, 0), (0, HSTRIDE - WIN))).reshape(N_HEADS * HSTRIDE)

    mesh = plsc.VectorSubcoreMesh(core_axis_name="c", subcore_axis_name="s")
    f = pl.kernel(
        _body,
        out_type=jax.ShapeDtypeStruct((N_HEADS, NQ, NK), jnp.float32),
        mesh=mesh,
        compiler_params=pltpu.CompilerParams(needs_layout_passes=False),
        scratch_types=[
            pltpu.VMEM((N_HEADS * HSTRIDE,), jnp.float32),
            pltpu.VMEM((NK,), jnp.int32),
            pltpu.VMEM((NK,), jnp.int32),
            pltpu.VMEM((NK,), jnp.int32),
            pltpu.VMEM((QPW,), jnp.int32),
            pltpu.VMEM((QPW,), jnp.int32),
            pltpu.VMEM((QPW,), jnp.int32),
            pltpu.VMEM((N_HEADS, 1, NK), jnp.float32),
            pltpu.VMEM((N_HEADS, 1, NK), jnp.float32),
            pltpu.SemaphoreType.DMA,
            pltpu.SemaphoreType.DMA,
            pltpu.SemaphoreType.DMA,
        ],
    )
    return f(qx, qy, kx, ky, tab)
